# Initial kernel scaffold; baseline (speedup 1.0000x reference)
import jax
import jax.numpy as jnp
from jax.experimental import pallas as pl


def kernel(x, edge_index, pseudo, W1, root1, b1, W2, root2, b2, W3, root3, b3, Wl1, bl1, Wl2, bl2):
    raise NotImplementedError("write your pallas kernel here")


# speedup vs baseline: 1.6952x; 1.6952x over previous
import functools

import jax
import jax.numpy as jnp
from jax import lax
from jax.experimental import pallas as pl
from jax.experimental.pallas import tpu as pltpu
from jax.experimental.pallas import tpu_sc as plsc

KS = (3, 4, 3)
KTOT = 36
NC, NS, LANES = 2, 16, 16
NW = NC * NS
BE = 64
NPAD = 6912


def _cdiv(a, b):
    return (a + b - 1) // b


def _prep_body(E, BEP, pseudo_ref, src_ref, wgt_ref, gidx_ref, val_ref):
    p0 = pseudo_ref[:, 0:1]
    p1 = pseudo_ref[:, 1:2]
    p2 = pseudo_ref[:, 2:3]
    u0 = p0 * 2.0
    i0a = jnp.clip(jnp.floor(u0), 0.0, 1.0)
    f0 = u0 - i0a
    u1 = p1 * 4.0
    i1a = jnp.clip(jnp.floor(u1), 0.0, 3.0)
    f1 = u1 - i1a
    u2 = p2 * 2.0
    i2a = jnp.clip(jnp.floor(u2), 0.0, 1.0)
    f2 = u2 - i2a
    i0a_i = i0a.astype(jnp.int32)
    i1a_i = i1a.astype(jnp.int32)
    i2a_i = i2a.astype(jnp.int32)
    i0b_i = i0a_i + 1
    i1b_i = jnp.where(i1a_i == 3, 0, i1a_i + 1)
    i2b_i = i2a_i + 1
    src = src_ref[:, 0:1]
    wcols = []
    kcols = []
    for b in range(8):
        b0, b1, b2 = b & 1, (b >> 1) & 1, (b >> 2) & 1
        w = (f0 if b0 else (1.0 - f0)) \
            * (f1 if b1 else (1.0 - f1)) \
            * (f2 if b2 else (1.0 - f2))
        i0 = i0b_i if b0 else i0a_i
        i1 = i1b_i if b1 else i1a_i
        i2 = i2b_i if b2 else i2a_i
        kidx = (i0 * 4 + i1) * 3 + i2
        wcols.append(w)
        kcols.append(src * KTOT + kidx)
    e_glob = pl.program_id(0) * BEP + lax.broadcasted_iota(
        jnp.int32, (BEP, 1), 0)
    valid = (e_glob < E).astype(jnp.float32)
    wgt_ref[...] = jnp.concatenate(wcols, axis=1) * valid
    gidx_ref[...] = jnp.concatenate(kcols, axis=1)
    col0 = (lax.broadcasted_iota(jnp.int32, (1, 16), 1) == 0)
    val_ref[...] = valid * col0.astype(jnp.float32)


def _edge_prep(pseudo_p, src_p, E):
    EPAD = pseudo_p.shape[0]
    BEP = 1024
    grid = EPAD // BEP
    return pl.pallas_call(
        functools.partial(_prep_body, E, BEP),
        grid=(grid,),
        in_specs=[
            pl.BlockSpec((BEP, 3), lambda i: (i, 0)),
            pl.BlockSpec((BEP, 1), lambda i: (i, 0)),
        ],
        out_specs=[
            pl.BlockSpec((BEP, 8), lambda i: (i, 0)),
            pl.BlockSpec((BEP, 8), lambda i: (i, 0)),
            pl.BlockSpec((BEP, 16), lambda i: (i, 0)),
        ],
        out_shape=[
            jax.ShapeDtypeStruct((EPAD, 8), jnp.float32),
            jax.ShapeDtypeStruct((EPAD, 8), jnp.int32),
            jax.ShapeDtypeStruct((EPAD, 16), jnp.float32),
        ],
    )(pseudo_p, src_p)


def _mm_body(h_ref, w_ref, o_ref):
    o_ref[...] = jnp.dot(h_ref[...], w_ref[...],
                         preferred_element_type=jnp.float32)


def _mm(h, w, br=256):
    n, k = h.shape
    m = w.shape[1]
    return pl.pallas_call(
        _mm_body,
        grid=(n // br,),
        in_specs=[
            pl.BlockSpec((br, k), lambda i: (i, 0)),
            pl.BlockSpec((k, m), lambda i: (0, 0)),
        ],
        out_specs=pl.BlockSpec((br, m), lambda i: (i, 0)),
        out_shape=jax.ShapeDtypeStruct((n, m), jnp.float32),
    )(h, w)


def _combine_body(a0_ref, a1_ref, d0_ref, d1_ref, h_ref, r_ref, b_ref, o_ref):
    agg = a0_ref[...] + a1_ref[...]
    deg = d0_ref[:, 0:1] + d1_ref[:, 0:1]
    root = jnp.dot(h_ref[...], r_ref[...], preferred_element_type=jnp.float32)
    o_ref[...] = jnp.maximum(
        agg / jnp.maximum(deg, 1.0) + root + b_ref[...], 0.0)


def _combine(a0, a1, d0, d1, h, root, bias):
    n, fo = a0.shape
    k = h.shape[1]
    br = 256
    b2 = bias.reshape(1, fo)
    return pl.pallas_call(
        _combine_body,
        grid=(n // br,),
        in_specs=[
            pl.BlockSpec((br, fo), lambda i: (i, 0)),
            pl.BlockSpec((br, fo), lambda i: (i, 0)),
            pl.BlockSpec((br, 16), lambda i: (i, 0)),
            pl.BlockSpec((br, 16), lambda i: (i, 0)),
            pl.BlockSpec((br, k), lambda i: (i, 0)),
            pl.BlockSpec((k, fo), lambda i: (0, 0)),
            pl.BlockSpec((1, fo), lambda i: (0, 0)),
        ],
        out_specs=pl.BlockSpec((br, fo), lambda i: (i, 0)),
        out_shape=jax.ShapeDtypeStruct((n, fo), jnp.float32),
    )(a0, a1, d0, d1, h, root, b2)


def _head1_body(h_ref, w_ref, b_ref, o_ref):
    o_ref[...] = jnp.maximum(
        jnp.dot(h_ref[...], w_ref[...], preferred_element_type=jnp.float32)
        + b_ref[...], 0.0)


def _head1(h, w, bias):
    n, k = h.shape
    m = w.shape[1]
    br = 256
    return pl.pallas_call(
        _head1_body,
        grid=(n // br,),
        in_specs=[
            pl.BlockSpec((br, k), lambda i: (i, 0)),
            pl.BlockSpec((k, m), lambda i: (0, 0)),
            pl.BlockSpec((1, m), lambda i: (0, 0)),
        ],
        out_specs=pl.BlockSpec((br, m), lambda i: (i, 0)),
        out_shape=jax.ShapeDtypeStruct((n, m), jnp.float32),
    )(h, w, bias.reshape(1, m))


def _head2_body(h_ref, w_ref, b_ref, o_ref):
    logits = jnp.dot(h_ref[...], w_ref[...],
                     preferred_element_type=jnp.float32) + b_ref[...]
    m = jnp.max(logits, axis=-1, keepdims=True)
    lse = jnp.log(jnp.sum(jnp.exp(logits - m), axis=-1, keepdims=True))
    o_ref[...] = logits - m - lse


def _head2(h, w, bias):
    n, k = h.shape
    m = w.shape[1]
    br = 128
    return pl.pallas_call(
        _head2_body,
        grid=(n // br,),
        in_specs=[
            pl.BlockSpec((br, k), lambda i: (i, 0)),
            pl.BlockSpec((k, m), lambda i: (0, 0)),
            pl.BlockSpec((1, m), lambda i: (0, 0)),
        ],
        out_specs=pl.BlockSpec((br, m), lambda i: (i, 0)),
        out_shape=jax.ShapeDtypeStruct((n, m), jnp.float32),
    )(h, w, bias.reshape(1, m))


def _sc_conv_body(F, KB, y_hbm, gidx_hbm, wgt_hbm, dst_hbm, z_hbm, out_hbm,
                  gidx_v, dst_v, wgt_v, rows_v, msg_v, agg_sh, sem):
    c = lax.axis_index("c")
    s = lax.axis_index("s")
    w = s * NC + c
    rp = NPAD // NS
    pltpu.sync_copy(z_hbm.at[pl.ds(s * rp, rp)], agg_sh.at[pl.ds(s * rp, rp)])
    pltpu.sync_copy(gidx_hbm.at[w], gidx_v)
    pltpu.sync_copy(dst_hbm.at[w], dst_v)
    pltpu.sync_copy(wgt_hbm.at[w], wgt_v)
    plsc.subcore_barrier()
    nv = F // LANES

    def block(kb, carry):
        for half in range(2):
            for j in range(2):
                pltpu.async_copy(y_hbm.at[gidx_v.at[kb * 4 + half * 2 + j]],
                                 rows_v.at[pl.ds(j * 128, 128)], sem).wait()

            def edge(e, cc):
                base = kb * 512 + half * 256 + e * 8
                accs = [jnp.zeros((LANES,), jnp.float32)] * nv
                for b in range(8):
                    wb = plsc.load_gather(
                        wgt_v, [jnp.full((LANES,), base + b, jnp.int32)])
                    for j2 in range(nv):
                        accs[j2] = accs[j2] + wb * rows_v[e * 8 + b,
                                                          pl.ds(j2 * 16, 16)]
                for j2 in range(nv):
                    msg_v[half * 32 + e, pl.ds(j2 * 16, 16)] = accs[j2]
                return cc

            lax.fori_loop(0, 32, edge, 0)
        pltpu.sync_copy(msg_v, agg_sh.at[dst_v.at[kb]], add=True)
        return carry

    lax.fori_loop(0, KB, block, 0)
    plsc.subcore_barrier()
    pltpu.sync_copy(agg_sh.at[pl.ds(s * rp, rp)],
                    out_hbm.at[c].at[pl.ds(s * rp, rp)])


def _sc_conv(y2d, gidx_sc, wgt_sc, dst_sc, zeros, F, KB):
    mesh = plsc.VectorSubcoreMesh(core_axis_name="c", subcore_axis_name="s")
    fn = pl.kernel(
        functools.partial(_sc_conv_body, F, KB),
        out_type=jax.ShapeDtypeStruct((NC, NPAD, F), jnp.float32),
        mesh=mesh,
        scratch_types=[
            pltpu.VMEM((KB * 4, 128), jnp.int32),
            pltpu.VMEM((KB, BE), jnp.int32),
            pltpu.VMEM((KB * 512,), jnp.float32),
            pltpu.VMEM((256, F), jnp.float32),
            pltpu.VMEM((BE, F), jnp.float32),
            pltpu.VMEM_SHARED((NPAD, F), jnp.float32),
            pltpu.SemaphoreType.DMA,
        ],
        compiler_params=pltpu.CompilerParams(use_tc_tiling_on_sc=False, needs_layout_passes=False),
    )
    return fn(y2d, gidx_sc, wgt_sc, dst_sc, zeros)


def _sc_deg_body(KB, val_hbm, dst_hbm, z_hbm, out_hbm, val_v, dst_v, deg_sh):
    c = lax.axis_index("c")
    s = lax.axis_index("s")
    w = s * NC + c
    rp = NPAD // NS
    pltpu.sync_copy(z_hbm.at[pl.ds(s * rp, rp)], deg_sh.at[pl.ds(s * rp, rp)])
    pltpu.sync_copy(val_hbm.at[w], val_v)
    pltpu.sync_copy(dst_hbm.at[w], dst_v)
    plsc.subcore_barrier()

    def block(kb, carry):
        pltpu.sync_copy(val_v.at[kb], deg_sh.at[dst_v.at[kb]], add=True)
        return carry

    lax.fori_loop(0, KB, block, 0)
    plsc.subcore_barrier()
    pltpu.sync_copy(deg_sh.at[pl.ds(s * rp, rp)],
                    out_hbm.at[c].at[pl.ds(s * rp, rp)])


def _sc_deg(val_sc, dst_sc, zeros, KB):
    mesh = plsc.VectorSubcoreMesh(core_axis_name="c", subcore_axis_name="s")
    fn = pl.kernel(
        functools.partial(_sc_deg_body, KB),
        out_type=jax.ShapeDtypeStruct((NC, NPAD, 16), jnp.float32),
        mesh=mesh,
        scratch_types=[
            pltpu.VMEM((KB, BE, 16), jnp.float32),
            pltpu.VMEM((KB, BE), jnp.int32),
            pltpu.VMEM_SHARED((NPAD, 16), jnp.float32),
        ],
        compiler_params=pltpu.CompilerParams(use_tc_tiling_on_sc=False, needs_layout_passes=False),
    )
    return fn(val_sc, dst_sc, zeros)


def _conv_layer(h, wflat, gidx_sc, wgt_sc, dst_sc, d0, d1, root_p, bias, F,
                KB, zeros):
    y = _mm(h, wflat)
    y2d = y.reshape(NPAD * KTOT, F)
    agg = _sc_conv(y2d, gidx_sc, wgt_sc, dst_sc, zeros, F, KB)
    return _combine(agg[0], agg[1], d0, d1, h, root_p, bias)


def kernel(x, edge_index, pseudo, W1, root1, b1, W2, root2, b2, W3, root3, b3,
           Wl1, bl1, Wl2, bl2):
    N = x.shape[0]
    E = edge_index.shape[1]
    KB = _cdiv(E, NW * BE)
    EPAD = NW * BE * KB

    src = edge_index[0]
    dst = edge_index[1]
    pseudo_p = jnp.pad(pseudo, ((0, EPAD - E), (0, 0)))
    src_p = jnp.pad(src, (0, EPAD - E)).reshape(EPAD, 1)
    dst_p = jnp.pad(dst, (0, EPAD - E))

    wgt8, gidx8, valid16 = _edge_prep(pseudo_p, src_p, E)

    gidx_sc = gidx8.reshape(NW, KB * 4, 128)
    wgt_sc = wgt8.reshape(NW, KB * 512)
    dst_sc = dst_p.reshape(NW, KB, BE)
    val_sc = valid16.reshape(NW, KB, BE, 16)

    zeros16 = jnp.zeros((NPAD, 16), jnp.float32)
    deg = _sc_deg(val_sc, dst_sc, zeros16, KB)
    d0, d1 = deg[0], deg[1]

    xp = jnp.pad(x, ((0, NPAD - N), (0, 7)))

    w1f = jnp.pad(W1.transpose(1, 0, 2).reshape(1, KTOT * 32), ((0, 7), (0, 0)))
    r1p = jnp.pad(root1, ((0, 7), (0, 0)))
    w2f = W2.transpose(1, 0, 2).reshape(32, KTOT * 64)
    w3f = W3.transpose(1, 0, 2).reshape(64, KTOT * 128)

    z32 = jnp.zeros((NPAD, 32), jnp.float32)
    z64 = jnp.zeros((NPAD, 64), jnp.float32)
    z128 = jnp.zeros((NPAD, 128), jnp.float32)

    h1 = _conv_layer(xp, w1f, gidx_sc, wgt_sc, dst_sc, d0, d1, r1p, b1, 32,
                     KB, z32)
    h2 = _conv_layer(h1, w2f, gidx_sc, wgt_sc, dst_sc, d0, d1, root2, b2, 64,
                     KB, z64)
    h3 = _conv_layer(h2, w3f, gidx_sc, wgt_sc, dst_sc, d0, d1, root3, b3, 128,
                     KB, z128)

    h4 = _head1(h3, Wl1, bl1)
    ncls = Wl2.shape[1]
    cpad = NPAD - ncls
    wl2p = jnp.pad(Wl2, ((0, 0), (0, cpad)))
    bl2p = jnp.pad(bl2, (0, cpad), constant_values=-1e30)
    out = _head2(h4, wl2p, bl2p)
    return out[:N, :ncls]
